# baseline (speedup 1.0000x reference)
import jax
import jax.numpy as jnp
from jax.experimental import pallas as pl
from jax.experimental.pallas import tpu as pltpu

_BLK_ROWS = 1024


def _copy_rmw(key_ref, cache_ref, out_ref):
    out_ref[...] = cache_ref[...]
    i = pl.program_id(0)
    j = pl.program_id(1)

    @pl.when((i == 1) & (j == 0))
    def _():
        tile = out_ref[0, 0:8, 0:128]
        rows = jax.lax.broadcasted_iota(jnp.int32, tile.shape, 0)
        cols = jax.lax.broadcasted_iota(jnp.int32, tile.shape, 1)
        upd = jnp.where((rows == 0) & (cols == 1), 2.0 * key_ref[0], 0.0)
        out_ref[0, 0:8, 0:128] = tile + upd


def kernel(key, cache_next):
    d0, d1, d2 = cache_next.shape
    updated = pl.pallas_call(
        _copy_rmw,
        grid=(d0, d1 // _BLK_ROWS),
        in_specs=[
            pl.BlockSpec(memory_space=pltpu.SMEM),
            pl.BlockSpec((1, _BLK_ROWS, d2), lambda i, j: (i, j, 0)),
        ],
        out_specs=pl.BlockSpec((1, _BLK_ROWS, d2), lambda i, j: (i, j, 0)),
        out_shape=jax.ShapeDtypeStruct(cache_next.shape, cache_next.dtype),
    )(key, cache_next)
    return key, updated
